# Initial kernel scaffold; baseline (speedup 1.0000x reference)
import jax
import jax.numpy as jnp
from jax.experimental import pallas as pl


def kernel(x_user, x_artwork, edge_index_u2a, edge_index_a2u, edge_weight_u2a, edge_weight_a2u, entries, Wl_u2a, Wr_u2a, b_u2a, Wl_a2u, Wr_a2u, b_a2u, lin_user_w, lin_user_b, lin_item_w, lin_item_b, out_w, out_b):
    raise NotImplementedError("write your pallas kernel here")


# speedup vs baseline: 2.3776x; 2.3776x over previous
import functools

import jax
import jax.numpy as jnp
from jax import lax
from jax.experimental import pallas as pl
from jax.experimental.pallas import tpu as pltpu
from jax.experimental.pallas import tpu_sc as plsc

NC = 2
NS = 16
NW = NC * NS
EC = 128
CP = 64
F32 = jnp.float32


def _sc_mesh():
    return plsc.VectorSubcoreMesh(
        core_axis_name="c", subcore_axis_name="s", num_cores=NC,
        num_subcores=NS)


def _splat_lane(v16, lane):
    idx = jnp.full((16, 1), lane, jnp.int32)
    dn = lax.GatherDimensionNumbers(
        offset_dims=(), collapsed_slice_dims=(0,), start_index_map=(0,))
    return lax.gather(v16, idx, dn, (1,),
                      mode=lax.GatherScatterMode.PROMISE_IN_BOUNDS)


def _make_segsum(n_src, n_pad, w, ch):
    rows_pt = n_pad // NS
    n_cp = rows_pt // CP

    @functools.partial(
        pl.kernel,
        out_type=jax.ShapeDtypeStruct((NC, n_pad, w), F32),
        mesh=_sc_mesh(),
        compiler_params=pltpu.CompilerParams(use_tc_tiling_on_sc=False),
        scratch_types=[
            pltpu.VMEM((ch, EC), jnp.int32),
            pltpu.VMEM((ch, EC), jnp.int32),
            pltpu.VMEM((EC,), F32),
            pltpu.VMEM((EC, w), F32),
            pltpu.VMEM_SHARED((n_pad, w), F32),
        ],
    )
    def k(x_hbm, src_hbm, dst_hbm, ew_hbm, out_hbm, src_v, dst_v, ew_v,
          msg_v, acc):
        cid = lax.axis_index("c")
        sid = lax.axis_index("s")
        wid = sid * NC + cid

        def zrow(r, _):
            for cc in range(w // 16):
                msg_v[r, pl.ds(cc * 16, 16)] = jnp.zeros((16,), F32)
            return 0
        lax.fori_loop(0, EC, zrow, 0)

        base0 = sid * rows_pt

        def zacc(kk, _):
            pltpu.sync_copy(msg_v.at[pl.ds(0, CP)],
                            acc.at[pl.ds(base0 + kk * CP, CP)])
            return 0
        lax.fori_loop(0, n_cp, zacc, 0)

        pltpu.sync_copy(src_hbm.at[wid], src_v)
        pltpu.sync_copy(dst_hbm.at[wid], dst_v)
        plsc.subcore_barrier()

        def chunk(j, _):
            pltpu.sync_copy(ew_hbm.at[wid, j], ew_v)
            pltpu.sync_copy(x_hbm.at[src_v.at[j]], msg_v)

            def scale16(g, _):
                wrow = ew_v[pl.ds(g * 16, 16)]

                def one(e16, _):
                    w16 = _splat_lane(wrow, e16)
                    row = g * 16 + e16
                    for cc in range(w // 16):
                        sl = pl.ds(cc * 16, 16)
                        msg_v[row, sl] = msg_v[row, sl] * w16
                    return 0
                lax.fori_loop(0, 16, one, 0)
                return 0
            lax.fori_loop(0, EC // 16, scale16, 0)

            pltpu.sync_copy(msg_v, acc.at[dst_v.at[j]], add=True)
            return 0
        lax.fori_loop(0, ch, chunk, 0)

        plsc.subcore_barrier()

        def cpout(kk, _):
            b = base0 + kk * CP
            pltpu.sync_copy(acc.at[pl.ds(b, CP)], msg_v.at[pl.ds(0, CP)])
            pltpu.sync_copy(msg_v.at[pl.ds(0, CP)],
                            out_hbm.at[cid, pl.ds(b, CP)])
            return 0
        lax.fori_loop(0, n_cp, cpout, 0)

    return k


def _make_counts(n_pad, ch):
    w = 16
    rows_pt = n_pad // NS
    n_cp = rows_pt // CP

    @functools.partial(
        pl.kernel,
        out_type=jax.ShapeDtypeStruct((NC, n_pad, w), F32),
        mesh=_sc_mesh(),
        compiler_params=pltpu.CompilerParams(use_tc_tiling_on_sc=False),
        scratch_types=[
            pltpu.VMEM((ch, EC), jnp.int32),
            pltpu.VMEM((EC, w), F32),
            pltpu.VMEM_SHARED((n_pad, w), F32),
        ],
    )
    def k(dst_hbm, out_hbm, dst_v, ones_v, acc):
        cid = lax.axis_index("c")
        sid = lax.axis_index("s")
        wid = sid * NC + cid

        def zrow(r, _):
            ones_v[r, pl.ds(0, 16)] = jnp.zeros((16,), F32)
            return 0
        lax.fori_loop(0, EC, zrow, 0)

        base0 = sid * rows_pt

        def zacc(kk, _):
            pltpu.sync_copy(ones_v.at[pl.ds(0, CP)],
                            acc.at[pl.ds(base0 + kk * CP, CP)])
            return 0
        lax.fori_loop(0, n_cp, zacc, 0)

        pltpu.sync_copy(dst_hbm.at[wid], dst_v)

        def orow(r, _):
            ones_v[r, pl.ds(0, 16)] = jnp.ones((16,), F32)
            return 0
        lax.fori_loop(0, EC, orow, 0)
        plsc.subcore_barrier()

        def chunk(j, _):
            pltpu.sync_copy(ones_v, acc.at[dst_v.at[j]], add=True)
            return 0
        lax.fori_loop(0, ch, chunk, 0)

        plsc.subcore_barrier()

        def cpout(kk, _):
            b = base0 + kk * CP
            pltpu.sync_copy(acc.at[pl.ds(b, CP)], ones_v.at[pl.ds(0, CP)])
            pltpu.sync_copy(ones_v.at[pl.ds(0, CP)],
                            out_hbm.at[cid, pl.ds(b, CP)])
            return 0
        lax.fori_loop(0, n_cp, cpout, 0)

    return k


def _make_pair_gather(n_u, n_a, b, d):
    bpw = b // NW

    @functools.partial(
        pl.kernel,
        out_type=(jax.ShapeDtypeStruct((b, d), F32),
                  jax.ShapeDtypeStruct((b, d), F32)),
        mesh=_sc_mesh(),
        compiler_params=pltpu.CompilerParams(use_tc_tiling_on_sc=False),
        scratch_types=[
            pltpu.VMEM((1, bpw), jnp.int32),
            pltpu.VMEM((bpw, d), F32),
        ],
    )
    def k(hu_hbm, ha_hbm, eu_hbm, ea_hbm, ou_hbm, oa_hbm, idx_v, rows_v):
        cid = lax.axis_index("c")
        sid = lax.axis_index("s")
        wid = sid * NC + cid
        base = wid * bpw
        pltpu.sync_copy(eu_hbm.at[pl.ds(base, bpw)], idx_v.at[0])
        pltpu.sync_copy(hu_hbm.at[idx_v.at[0]], rows_v)
        pltpu.sync_copy(rows_v, ou_hbm.at[pl.ds(base, bpw)])
        pltpu.sync_copy(ea_hbm.at[pl.ds(base, bpw)], idx_v.at[0])
        pltpu.sync_copy(ha_hbm.at[idx_v.at[0]], rows_v)
        pltpu.sync_copy(rows_v, oa_hbm.at[pl.ds(base, bpw)])

    return k


def _layer_tc(agg0, agg1, cnt0, cnt1, h, wl, wr, bias, relu):
    n, d = h.shape
    rb = 400

    def body(a0, a1, c0, c1, hh, wlr, wrr, br, o):
        cnt = jnp.maximum(c0[...] + c1[...], 1.0)
        agg = (a0[...] + a1[...]) / cnt
        y = (jnp.dot(agg, wlr[...], preferred_element_type=F32)
             + jnp.dot(hh[...], wrr[...], preferred_element_type=F32)
             + br[...][None, :])
        o[...] = jnp.maximum(y, 0.0) if relu else y

    grid = (n // rb,)
    mat = pl.BlockSpec((rb, d), lambda i: (i, 0))
    vec = pl.BlockSpec((rb, 1), lambda i: (i, 0))
    wsp = pl.BlockSpec((d, d), lambda i: (0, 0))
    bsp = pl.BlockSpec((d,), lambda i: (0,))
    return pl.pallas_call(
        body,
        grid=grid,
        in_specs=[mat, mat, vec, vec, mat, wsp, wsp, bsp],
        out_specs=mat,
        out_shape=jax.ShapeDtypeStruct((n, d), F32),
    )(agg0, agg1, cnt0, cnt1, h, wl, wr, bias)


def _head_tc(u, it, luw, lub, liw, lib, ow, ob):
    b, d = u.shape
    hd = d // 2

    def body(ur, ir, luwr, lubr, liwr, libr, owr, obr, o):
        uf = (jnp.dot(ur[...], luwr[...], preferred_element_type=F32)
              + lubr[...][None, :])
        itf = (jnp.dot(ir[...], liwr[...], preferred_element_type=F32)
               + libr[...][None, :])
        s = (jnp.dot(uf, owr[0:hd, :], preferred_element_type=F32)
             + jnp.dot(itf, owr[hd:d, :], preferred_element_type=F32)
             + obr[...][None, :])
        o[...] = s

    return pl.pallas_call(
        body,
        out_shape=jax.ShapeDtypeStruct((b, 1), F32),
    )(u, it, luw, lub, liw, lib, ow, ob)


def kernel(x_user, x_artwork, edge_index_u2a, edge_index_a2u,
           edge_weight_u2a, edge_weight_a2u, entries,
           Wl_u2a, Wr_u2a, b_u2a, Wl_a2u, Wr_a2u, b_a2u,
           lin_user_w, lin_user_b, lin_item_w, lin_item_b, out_w, out_b):
    n_user, d = x_user.shape
    n_art = x_artwork.shape[0]
    e = edge_index_u2a.shape[1]
    bsz = entries.shape[1]

    slab = NW * EC
    e_pad = ((e + slab - 1) // slab) * slab
    ch = e_pad // slab
    up = ((n_user + NS * CP - 1) // (NS * CP)) * (NS * CP)
    ap = ((n_art + NS * CP - 1) // (NS * CP)) * (NS * CP)

    def prep(ei, ew, n_dst):
        pad = e_pad - e
        src = jnp.concatenate([ei[0], jnp.zeros((pad,), jnp.int32)])
        dst = jnp.concatenate([ei[1], jnp.full((pad,), n_dst, jnp.int32)])
        eww = jnp.concatenate([ew, jnp.zeros((pad,), F32)])
        return (src.reshape(NW, ch, EC), dst.reshape(NW, ch, EC),
                eww.reshape(NW, ch, EC))

    src_ua, dst_ua, ew_ua = prep(edge_index_u2a, edge_weight_u2a, n_art)
    src_au, dst_au, ew_au = prep(edge_index_a2u, edge_weight_a2u, n_user)

    counts_a = _make_counts(ap, ch)(dst_ua)
    counts_u = _make_counts(up, ch)(dst_au)
    cnt_a0 = counts_a[0, :n_art, 0:1]
    cnt_a1 = counts_a[1, :n_art, 0:1]
    cnt_u0 = counts_u[0, :n_user, 0:1]
    cnt_u1 = counts_u[1, :n_user, 0:1]

    seg_a = _make_segsum(n_user, ap, d, ch)
    seg_u = _make_segsum(n_art, up, 32, ch)
    nchunk = d // 32

    def segmean_partials(h_u, h_a):
        agg_a = seg_a(h_u, src_ua, dst_ua, ew_ua)
        a0, a1 = agg_a[0, :n_art, :], agg_a[1, :n_art, :]
        u0c, u1c = [], []
        for j in range(nchunk):
            hc = h_a[:, j * 32:(j + 1) * 32]
            agg_u = seg_u(hc, src_au, dst_au, ew_au)
            u0c.append(agg_u[0, :n_user, :])
            u1c.append(agg_u[1, :n_user, :])
        return a0, a1, jnp.concatenate(u0c, 1), jnp.concatenate(u1c, 1)

    h_u, h_a = x_user, x_artwork
    n_layers = Wl_u2a.shape[0]
    for l in range(n_layers):
        relu = l < n_layers - 1
        a0, a1, u0, u1 = segmean_partials(h_u, h_a)
        new_a = _layer_tc(a0, a1, cnt_a0, cnt_a1, h_a,
                          Wl_u2a[l], Wr_u2a[l], b_u2a[l], relu)
        new_u = _layer_tc(u0, u1, cnt_u0, cnt_u1, h_u,
                          Wl_a2u[l], Wr_a2u[l], b_a2u[l], relu)
        h_u, h_a = new_u, new_a

    rows_u, rows_a = _make_pair_gather(n_user, n_art, bsz, d)(
        h_u, h_a, entries[0], entries[1])

    return _head_tc(rows_u, rows_a, lin_user_w, lin_user_b,
                    lin_item_w, lin_item_b, out_w, out_b)


# speedup vs baseline: 2.6086x; 1.0972x over previous
import functools

import jax
import jax.numpy as jnp
from jax import lax
from jax.experimental import pallas as pl
from jax.experimental.pallas import tpu as pltpu
from jax.experimental.pallas import tpu_sc as plsc

NC = 2
NS = 16
NW = NC * NS
EC = 128
CP = 64
F32 = jnp.float32


def _sc_mesh():
    return plsc.VectorSubcoreMesh(
        core_axis_name="c", subcore_axis_name="s", num_cores=NC,
        num_subcores=NS)


def _splat_lane(v16, lane):
    idx = jnp.full((16, 1), lane, jnp.int32)
    dn = lax.GatherDimensionNumbers(
        offset_dims=(), collapsed_slice_dims=(0,), start_index_map=(0,))
    return lax.gather(v16, idx, dn, (1,),
                      mode=lax.GatherScatterMode.PROMISE_IN_BOUNDS)


def _make_segsum(n_src, n_pad, w, ch):
    rows_pt = n_pad // NS
    n_cp = rows_pt // CP

    @functools.partial(
        pl.kernel,
        out_type=jax.ShapeDtypeStruct((NC, n_pad, w), F32),
        mesh=_sc_mesh(),
        compiler_params=pltpu.CompilerParams(use_tc_tiling_on_sc=False),
        scratch_types=[
            pltpu.VMEM((ch, EC), jnp.int32),
            pltpu.VMEM((ch, EC), jnp.int32),
            pltpu.VMEM((EC,), F32),
            pltpu.VMEM((EC, w), F32),
            pltpu.VMEM_SHARED((n_pad, w), F32),
        ],
    )
    def k(x_hbm, src_hbm, dst_hbm, ew_hbm, out_hbm, src_v, dst_v, ew_v,
          msg_v, acc):
        cid = lax.axis_index("c")
        sid = lax.axis_index("s")
        wid = sid * NC + cid

        def zrow(r, _):
            for cc in range(w // 16):
                msg_v[r, pl.ds(cc * 16, 16)] = jnp.zeros((16,), F32)
            return 0
        lax.fori_loop(0, EC, zrow, 0)

        base0 = sid * rows_pt

        def zacc(kk, _):
            pltpu.sync_copy(msg_v.at[pl.ds(0, CP)],
                            acc.at[pl.ds(base0 + kk * CP, CP)])
            return 0
        lax.fori_loop(0, n_cp, zacc, 0)

        pltpu.sync_copy(src_hbm.at[wid], src_v)
        pltpu.sync_copy(dst_hbm.at[wid], dst_v)
        plsc.subcore_barrier()

        def chunk(j, _):
            pltpu.sync_copy(ew_hbm.at[wid, j], ew_v)
            pltpu.sync_copy(x_hbm.at[src_v.at[j]], msg_v)

            def scale16(g, _):
                wrow = ew_v[pl.ds(g * 16, 16)]
                for e16 in range(16):
                    w16 = _splat_lane(wrow, e16)
                    row = g * 16 + e16
                    for cc in range(w // 16):
                        sl = pl.ds(cc * 16, 16)
                        msg_v[row, sl] = msg_v[row, sl] * w16
                return 0
            lax.fori_loop(0, EC // 16, scale16, 0)

            pltpu.sync_copy(msg_v, acc.at[dst_v.at[j]], add=True)
            return 0
        lax.fori_loop(0, ch, chunk, 0)

        plsc.subcore_barrier()

        def cpout(kk, _):
            b = base0 + kk * CP
            pltpu.sync_copy(acc.at[pl.ds(b, CP)], msg_v.at[pl.ds(0, CP)])
            pltpu.sync_copy(msg_v.at[pl.ds(0, CP)],
                            out_hbm.at[cid, pl.ds(b, CP)])
            return 0
        lax.fori_loop(0, n_cp, cpout, 0)

    return k


def _make_counts(n_pad, ch):
    w = 16
    rows_pt = n_pad // NS
    n_cp = rows_pt // CP

    @functools.partial(
        pl.kernel,
        out_type=jax.ShapeDtypeStruct((NC, n_pad, w), F32),
        mesh=_sc_mesh(),
        compiler_params=pltpu.CompilerParams(use_tc_tiling_on_sc=False),
        scratch_types=[
            pltpu.VMEM((ch, EC), jnp.int32),
            pltpu.VMEM((EC, w), F32),
            pltpu.VMEM_SHARED((n_pad, w), F32),
        ],
    )
    def k(dst_hbm, out_hbm, dst_v, ones_v, acc):
        cid = lax.axis_index("c")
        sid = lax.axis_index("s")
        wid = sid * NC + cid

        def zrow(r, _):
            ones_v[r, pl.ds(0, 16)] = jnp.zeros((16,), F32)
            return 0
        lax.fori_loop(0, EC, zrow, 0)

        base0 = sid * rows_pt

        def zacc(kk, _):
            pltpu.sync_copy(ones_v.at[pl.ds(0, CP)],
                            acc.at[pl.ds(base0 + kk * CP, CP)])
            return 0
        lax.fori_loop(0, n_cp, zacc, 0)

        pltpu.sync_copy(dst_hbm.at[wid], dst_v)

        def orow(r, _):
            ones_v[r, pl.ds(0, 16)] = jnp.ones((16,), F32)
            return 0
        lax.fori_loop(0, EC, orow, 0)
        plsc.subcore_barrier()

        def chunk(j, _):
            pltpu.sync_copy(ones_v, acc.at[dst_v.at[j]], add=True)
            return 0
        lax.fori_loop(0, ch, chunk, 0)

        plsc.subcore_barrier()

        def cpout(kk, _):
            b = base0 + kk * CP
            pltpu.sync_copy(acc.at[pl.ds(b, CP)], ones_v.at[pl.ds(0, CP)])
            pltpu.sync_copy(ones_v.at[pl.ds(0, CP)],
                            out_hbm.at[cid, pl.ds(b, CP)])
            return 0
        lax.fori_loop(0, n_cp, cpout, 0)

    return k


def _make_pair_gather(n_u, n_a, b, d):
    bpw = b // NW

    @functools.partial(
        pl.kernel,
        out_type=(jax.ShapeDtypeStruct((b, d), F32),
                  jax.ShapeDtypeStruct((b, d), F32)),
        mesh=_sc_mesh(),
        compiler_params=pltpu.CompilerParams(use_tc_tiling_on_sc=False),
        scratch_types=[
            pltpu.VMEM((1, bpw), jnp.int32),
            pltpu.VMEM((bpw, d), F32),
        ],
    )
    def k(hu_hbm, ha_hbm, eu_hbm, ea_hbm, ou_hbm, oa_hbm, idx_v, rows_v):
        cid = lax.axis_index("c")
        sid = lax.axis_index("s")
        wid = sid * NC + cid
        base = wid * bpw
        pltpu.sync_copy(eu_hbm.at[pl.ds(base, bpw)], idx_v.at[0])
        pltpu.sync_copy(hu_hbm.at[idx_v.at[0]], rows_v)
        pltpu.sync_copy(rows_v, ou_hbm.at[pl.ds(base, bpw)])
        pltpu.sync_copy(ea_hbm.at[pl.ds(base, bpw)], idx_v.at[0])
        pltpu.sync_copy(ha_hbm.at[idx_v.at[0]], rows_v)
        pltpu.sync_copy(rows_v, oa_hbm.at[pl.ds(base, bpw)])

    return k


def _layer_tc(agg0, agg1, cnt0, cnt1, h, wl, wr, bias, relu):
    n, d = h.shape
    rb = 400

    def body(a0, a1, c0, c1, hh, wlr, wrr, br, o):
        cnt = jnp.maximum(c0[...] + c1[...], 1.0)
        agg = (a0[...] + a1[...]) / cnt
        y = (jnp.dot(agg, wlr[...], preferred_element_type=F32)
             + jnp.dot(hh[...], wrr[...], preferred_element_type=F32)
             + br[...][None, :])
        o[...] = jnp.maximum(y, 0.0) if relu else y

    grid = (n // rb,)
    mat = pl.BlockSpec((rb, d), lambda i: (i, 0))
    vec = pl.BlockSpec((rb, 1), lambda i: (i, 0))
    wsp = pl.BlockSpec((d, d), lambda i: (0, 0))
    bsp = pl.BlockSpec((d,), lambda i: (0,))
    return pl.pallas_call(
        body,
        grid=grid,
        in_specs=[mat, mat, vec, vec, mat, wsp, wsp, bsp],
        out_specs=mat,
        out_shape=jax.ShapeDtypeStruct((n, d), F32),
    )(agg0, agg1, cnt0, cnt1, h, wl, wr, bias)


def _head_tc(u, it, luw, lub, liw, lib, ow, ob):
    b, d = u.shape
    hd = d // 2

    def body(ur, ir, luwr, lubr, liwr, libr, owr, obr, o):
        uf = (jnp.dot(ur[...], luwr[...], preferred_element_type=F32)
              + lubr[...][None, :])
        itf = (jnp.dot(ir[...], liwr[...], preferred_element_type=F32)
               + libr[...][None, :])
        s = (jnp.dot(uf, owr[0:hd, :], preferred_element_type=F32)
             + jnp.dot(itf, owr[hd:d, :], preferred_element_type=F32)
             + obr[...][None, :])
        o[...] = s

    return pl.pallas_call(
        body,
        out_shape=jax.ShapeDtypeStruct((b, 1), F32),
    )(u, it, luw, lub, liw, lib, ow, ob)


def kernel(x_user, x_artwork, edge_index_u2a, edge_index_a2u,
           edge_weight_u2a, edge_weight_a2u, entries,
           Wl_u2a, Wr_u2a, b_u2a, Wl_a2u, Wr_a2u, b_a2u,
           lin_user_w, lin_user_b, lin_item_w, lin_item_b, out_w, out_b):
    n_user, d = x_user.shape
    n_art = x_artwork.shape[0]
    e = edge_index_u2a.shape[1]
    bsz = entries.shape[1]

    slab = NW * EC
    e_pad = ((e + slab - 1) // slab) * slab
    ch = e_pad // slab
    up = ((n_user + NS * CP - 1) // (NS * CP)) * (NS * CP)
    ap = ((n_art + NS * CP - 1) // (NS * CP)) * (NS * CP)

    def prep(ei, ew, n_dst):
        pad = e_pad - e
        src = jnp.concatenate([ei[0], jnp.zeros((pad,), jnp.int32)])
        dst = jnp.concatenate([ei[1], jnp.full((pad,), n_dst, jnp.int32)])
        eww = jnp.concatenate([ew, jnp.zeros((pad,), F32)])
        return (src.reshape(NW, ch, EC), dst.reshape(NW, ch, EC),
                eww.reshape(NW, ch, EC))

    src_ua, dst_ua, ew_ua = prep(edge_index_u2a, edge_weight_u2a, n_art)
    src_au, dst_au, ew_au = prep(edge_index_a2u, edge_weight_a2u, n_user)

    counts_a = _make_counts(ap, ch)(dst_ua)
    counts_u = _make_counts(up, ch)(dst_au)
    cnt_a0 = counts_a[0, :n_art, 0:1]
    cnt_a1 = counts_a[1, :n_art, 0:1]
    cnt_u0 = counts_u[0, :n_user, 0:1]
    cnt_u1 = counts_u[1, :n_user, 0:1]

    seg_a = _make_segsum(n_user, ap, d, ch)
    seg_u = _make_segsum(n_art, up, 32, ch)
    nchunk = d // 32

    def segmean_partials(h_u, h_a):
        agg_a = seg_a(h_u, src_ua, dst_ua, ew_ua)
        a0, a1 = agg_a[0, :n_art, :], agg_a[1, :n_art, :]
        u0c, u1c = [], []
        for j in range(nchunk):
            hc = h_a[:, j * 32:(j + 1) * 32]
            agg_u = seg_u(hc, src_au, dst_au, ew_au)
            u0c.append(agg_u[0, :n_user, :])
            u1c.append(agg_u[1, :n_user, :])
        return a0, a1, jnp.concatenate(u0c, 1), jnp.concatenate(u1c, 1)

    h_u, h_a = x_user, x_artwork
    n_layers = Wl_u2a.shape[0]
    for l in range(n_layers):
        relu = l < n_layers - 1
        a0, a1, u0, u1 = segmean_partials(h_u, h_a)
        new_a = _layer_tc(a0, a1, cnt_a0, cnt_a1, h_a,
                          Wl_u2a[l], Wr_u2a[l], b_u2a[l], relu)
        new_u = _layer_tc(u0, u1, cnt_u0, cnt_u1, h_u,
                          Wl_a2u[l], Wr_a2u[l], b_a2u[l], relu)
        h_u, h_a = new_u, new_a

    rows_u, rows_a = _make_pair_gather(n_user, n_art, bsz, d)(
        h_u, h_a, entries[0], entries[1])

    return _head_tc(rows_u, rows_a, lin_user_w, lin_user_b,
                    lin_item_w, lin_item_b, out_w, out_b)
